# baseline (speedup 1.0000x reference)
import jax
import jax.numpy as jnp
from jax import lax
from jax.experimental import pallas as pl
from jax.experimental.pallas import tpu as pltpu
from jax.experimental.pallas import tpu_sc as plsc

B, L = 4096, 200
DW, DP = 32, 32
DO = DW + DP
N = B * L
NC, NS = 2, 16
NW = NC * NS
BW = B // NW
PER_W = BW * L
KB = 2
NGR = BW // KB
NBUF = 2
VLANES = 16


def _body(words_hbm, pos_hbm, ww_hbm, wp_hbm, out_hbm,
          idxw_v, idxp_v, rw_v, rp_v, semw, semp, semo):
    wid = lax.axis_index("s") * NC + lax.axis_index("c")
    base = wid * PER_W
    pltpu.sync_copy(words_hbm.at[pl.ds(base, PER_W)], idxw_v)
    pltpu.sync_copy(pos_hbm.at[pl.ds(base, PER_W)], idxp_v)
    b_base = wid * BW

    def scale(i, carry):
        s = pl.ds(i * VLANES, VLANES)
        idxw_v[s] = idxw_v[s] * 2
        return carry

    lax.fori_loop(0, PER_W // VLANES, scale, 0)

    def start_gather(b, j):
        for k in range(KB):
            r = j * KB + k
            pltpu.async_copy(ww_hbm.at[idxw_v.at[pl.ds(r * L, L)]],
                             rw_v.at[b, k], semw.at[b])
            pltpu.async_copy(wp_hbm.at[idxp_v.at[pl.ds(r * L, L)]],
                             rp_v.at[b, k], semp.at[b])

    def wait_gather(b, j):
        for k in range(KB):
            r = j * KB + k
            pltpu.make_async_copy(ww_hbm.at[idxw_v.at[pl.ds(r * L, L)]],
                                  rw_v.at[b, k], semw.at[b]).wait()
            pltpu.make_async_copy(wp_hbm.at[idxp_v.at[pl.ds(r * L, L)]],
                                  rp_v.at[b, k], semp.at[b]).wait()

    def start_write(b, j):
        b0 = b_base + j * KB
        pltpu.async_copy(rw_v.at[b],
                         out_hbm.at[pl.ds(b0, KB), :, pl.ds(0, DW)],
                         semo.at[b])
        pltpu.async_copy(rp_v.at[b],
                         out_hbm.at[pl.ds(b0, KB), :, pl.ds(DW, DP)],
                         semo.at[b])

    def wait_write(b, j):
        b0 = b_base + j * KB
        pltpu.make_async_copy(rw_v.at[b],
                              out_hbm.at[pl.ds(b0, KB), :, pl.ds(0, DW)],
                              semo.at[b]).wait()
        pltpu.make_async_copy(rp_v.at[b],
                              out_hbm.at[pl.ds(b0, KB), :, pl.ds(DW, DP)],
                              semo.at[b]).wait()

    for b in range(NBUF):
        start_gather(b, b)

    def step(it, carry):
        g = it * NBUF
        for b in range(NBUF):
            j = g + b
            wait_gather(b, j)
            start_write(b, j)
            wait_write(b, j)
            start_gather(b, j + NBUF)
        return carry

    lax.fori_loop(0, NGR // NBUF - 1, step, 0)

    for b in range(NBUF):
        j = NGR - NBUF + b
        wait_gather(b, j)
        start_write(b, j)
        wait_write(b, j)


@jax.jit
def _run(words_f, pos_f, W_words, W_pos):
    W4 = jnp.pad(W_words, ((0, 0), (0, 64 - DW))).reshape(2 * 1000000, DW)
    mesh = plsc.VectorSubcoreMesh(
        core_axis_name="c", subcore_axis_name="s",
        num_cores=NC, num_subcores=NS)
    f = pl.kernel(
        _body,
        out_type=jax.ShapeDtypeStruct((B, L, 128), jnp.float32),
        mesh=mesh,
        compiler_params=pltpu.CompilerParams(use_tc_tiling_on_sc=False),
        scratch_types=[
            pltpu.VMEM((PER_W,), jnp.int32),
            pltpu.VMEM((PER_W,), jnp.int32),
            pltpu.VMEM((NBUF, KB, L, DW), jnp.float32),
            pltpu.VMEM((NBUF, KB, L, DP), jnp.float32),
            pltpu.SemaphoreType.DMA((NBUF,)),
            pltpu.SemaphoreType.DMA((NBUF,)),
            pltpu.SemaphoreType.DMA((NBUF,)),
        ],
    )
    return f(words_f, pos_f, W4, W_pos)


def kernel(words, pos, W_words, W_pos):
    words_f = words.astype(jnp.int32).reshape(N)
    pos_f = pos.astype(jnp.int32).reshape(N)
    out = _run(words_f, pos_f, W_words, W_pos)
    return out[:, :, :DO]


# speedup vs baseline: 1.3964x; 1.3964x over previous
import jax
import jax.numpy as jnp
from jax import lax
from jax.experimental import pallas as pl
from jax.experimental.pallas import tpu as pltpu
from jax.experimental.pallas import tpu_sc as plsc

B, L = 4096, 200
DW, DP = 32, 32
DO = DW + DP
N = B * L
NC, NS = 2, 16
NW = NC * NS
BW = B // NW
PER_W = BW * L
KB = 1
NGR = BW // KB
NBUF = 4
VLANES = 16


def _body(words_hbm, pos_hbm, ww_hbm, wp_hbm, out_hbm,
          idxw_v, idxp_v, rw_v, rp_v, semw, semp, semo):
    wid = lax.axis_index("s") * NC + lax.axis_index("c")
    base = wid * PER_W
    pltpu.sync_copy(words_hbm.at[pl.ds(base, PER_W)], idxw_v)
    pltpu.sync_copy(pos_hbm.at[pl.ds(base, PER_W)], idxp_v)
    b_base = wid * BW

    def scale(i, carry):
        s = pl.ds(i * VLANES, VLANES)
        idxw_v[s] = idxw_v[s] * 4
        return carry

    lax.fori_loop(0, PER_W // VLANES, scale, 0)

    def start_gather(b, j):
        for k in range(KB):
            r = j * KB + k
            pltpu.async_copy(ww_hbm.at[idxw_v.at[pl.ds(r * L, L)]],
                             rw_v.at[b, k], semw.at[b])
            pltpu.async_copy(wp_hbm.at[idxp_v.at[pl.ds(r * L, L)]],
                             rp_v.at[b, k], semp.at[b])

    def wait_gather(b, j):
        for k in range(KB):
            r = j * KB + k
            pltpu.make_async_copy(ww_hbm.at[idxw_v.at[pl.ds(r * L, L)]],
                                  rw_v.at[b, k], semw.at[b]).wait()
            pltpu.make_async_copy(wp_hbm.at[idxp_v.at[pl.ds(r * L, L)]],
                                  rp_v.at[b, k], semp.at[b]).wait()

    def start_write(b, j):
        b0 = b_base + j * KB
        pltpu.async_copy(rw_v.at[b],
                         out_hbm.at[pl.ds(b0, KB), :, pl.ds(0, DW)],
                         semo.at[b])
        pltpu.async_copy(rp_v.at[b],
                         out_hbm.at[pl.ds(b0, KB), :, pl.ds(DW, DP)],
                         semo.at[b])

    def wait_write(b, j):
        b0 = b_base + j * KB
        pltpu.make_async_copy(rw_v.at[b],
                              out_hbm.at[pl.ds(b0, KB), :, pl.ds(0, DW)],
                              semo.at[b]).wait()
        pltpu.make_async_copy(rp_v.at[b],
                              out_hbm.at[pl.ds(b0, KB), :, pl.ds(DW, DP)],
                              semo.at[b]).wait()

    def steps12(s, j):
        wait_gather(s, j)
        start_write(s, j)

    def steps34(s2, j):
        wait_write(s2, j - 2)
        start_gather(s2, j + 2)

    start_gather(0, 0)
    start_gather(1, 1)
    steps12(0, 0)
    start_gather(2, 2)
    steps12(1, 1)
    start_gather(3, 3)
    steps12(2, 2)
    steps34(0, 2)
    steps12(3, 3)
    steps34(1, 3)

    def step(it, carry):
        g = it * 4
        for b in range(4):
            j = g + b
            steps12(b, j)
            steps34((b + 2) % 4, j)
        return carry

    lax.fori_loop(1, NGR // 4 - 1, step, 0)

    for b in range(4):
        j = NGR - 4 + b
        steps12(b, j)
        if j + 2 < NGR:
            steps34((b + 2) % 4, j)
    for b in range(4):
        j = NGR - 4 + b
        wait_write(b, j)


@jax.jit
def _run(words_f, pos_f, W_words, W_pos):
    W4 = jnp.pad(W_words, ((0, 0), (0, 128 - DW))).reshape(4 * 1000000, DW)
    mesh = plsc.VectorSubcoreMesh(
        core_axis_name="c", subcore_axis_name="s",
        num_cores=NC, num_subcores=NS)
    f = pl.kernel(
        _body,
        out_type=jax.ShapeDtypeStruct((B, L, 128), jnp.float32),
        mesh=mesh,
        compiler_params=pltpu.CompilerParams(use_tc_tiling_on_sc=False),
        scratch_types=[
            pltpu.VMEM((PER_W,), jnp.int32),
            pltpu.VMEM((PER_W,), jnp.int32),
            pltpu.VMEM((NBUF, KB, L, DW), jnp.float32),
            pltpu.VMEM((NBUF, KB, L, DP), jnp.float32),
            pltpu.SemaphoreType.DMA((NBUF,)),
            pltpu.SemaphoreType.DMA((NBUF,)),
            pltpu.SemaphoreType.DMA((NBUF,)),
        ],
    )
    return f(words_f, pos_f, W4, W_pos)


def kernel(words, pos, W_words, W_pos):
    words_f = words.astype(jnp.int32).reshape(N)
    pos_f = pos.astype(jnp.int32).reshape(N)
    out = _run(words_f, pos_f, W_words, W_pos)
    return out[:, :, :DO]


# speedup vs baseline: 1.4071x; 1.0077x over previous
import functools

import jax
import jax.numpy as jnp
from jax import lax
from jax.experimental import pallas as pl
from jax.experimental.pallas import tpu as pltpu
from jax.experimental.pallas import tpu_sc as plsc

B, L = 4096, 200
DW, DP = 32, 32
DO = DW + DP
N = B * L
NC, NS = 2, 16
NW = NC * NS
BW = B // NW
PER_W = BW * L
NGR = BW
VLANES = 16


def _gather_body(idx_scale, col_off, d,
                 idx_hbm, tbl_hbm, out_hbm, idx_v, rows_v, semg, semo):
    wid = lax.axis_index("s") * NC + lax.axis_index("c")
    base = wid * PER_W
    pltpu.sync_copy(idx_hbm.at[pl.ds(base, PER_W)], idx_v)
    b_base = wid * BW

    if idx_scale != 1:
        def scale(i, carry):
            s = pl.ds(i * VLANES, VLANES)
            idx_v[s] = idx_v[s] * idx_scale
            return carry
        lax.fori_loop(0, PER_W // VLANES, scale, 0)

    def start_gather(s, j):
        pltpu.async_copy(tbl_hbm.at[idx_v.at[pl.ds(j * L, L)]],
                         rows_v.at[s], semg.at[s])

    def wait_gather(s, j):
        pltpu.make_async_copy(tbl_hbm.at[idx_v.at[pl.ds(j * L, L)]],
                              rows_v.at[s], semg.at[s]).wait()

    def start_write(s, j):
        pltpu.async_copy(rows_v.at[s],
                         out_hbm.at[b_base + j, :, pl.ds(col_off, d)],
                         semo.at[s])

    def wait_write(s, j):
        pltpu.make_async_copy(rows_v.at[s],
                              out_hbm.at[b_base + j, :, pl.ds(col_off, d)],
                              semo.at[s]).wait()

    def steps12(s, j):
        wait_gather(s, j)
        start_write(s, j)

    def steps34(s2, j):
        wait_write(s2, j - 2)
        start_gather(s2, j + 2)

    start_gather(0, 0)
    start_gather(1, 1)
    steps12(0, 0)
    start_gather(2, 2)
    steps12(1, 1)
    start_gather(3, 3)
    steps12(2, 2)
    steps34(0, 2)
    steps12(3, 3)
    steps34(1, 3)

    def step(it, carry):
        g = it * 4
        for b in range(4):
            j = g + b
            steps12(b, j)
            steps34((b + 2) % 4, j)
        return carry

    lax.fori_loop(1, NGR // 4 - 1, step, 0)

    for b in range(4):
        j = NGR - 4 + b
        steps12(b, j)
        if j + 2 < NGR:
            steps34((b + 2) % 4, j)
    for b in range(4):
        wait_write(b, NGR - 4 + b)


def _mesh():
    return plsc.VectorSubcoreMesh(
        core_axis_name="c", subcore_axis_name="s",
        num_cores=NC, num_subcores=NS)


def _scratch(d):
    return [
        pltpu.VMEM((PER_W,), jnp.int32),
        pltpu.VMEM((4, L, d), jnp.float32),
        pltpu.SemaphoreType.DMA((4,)),
        pltpu.SemaphoreType.DMA((4,)),
    ]


@jax.jit
def _run(words_f, pos_f, W_words, W_pos):
    W4 = jnp.pad(W_words, ((0, 0), (0, 128 - DW))).reshape(4 * 1000000, DW)
    kpos = pl.kernel(
        functools.partial(_gather_body, 1, DW, DP),
        out_type=jax.ShapeDtypeStruct((B, L, 128), jnp.float32),
        mesh=_mesh(),
        compiler_params=pltpu.CompilerParams(use_tc_tiling_on_sc=False),
        scratch_types=_scratch(DP),
    )
    out1 = kpos(pos_f, W_pos)
    ref = jax.new_ref(out1)
    kwords = pl.kernel(
        functools.partial(_gather_body, 4, 0, DW),
        out_type=(),
        mesh=_mesh(),
        compiler_params=pltpu.CompilerParams(use_tc_tiling_on_sc=False),
        scratch_types=_scratch(DW),
    )
    kwords(words_f, W4, ref)
    return ref[...]


def kernel(words, pos, W_words, W_pos):
    words_f = words.astype(jnp.int32).reshape(N)
    pos_f = pos.astype(jnp.int32).reshape(N)
    out = _run(words_f, pos_f, W_words, W_pos)
    return out[:, :, :DO]


# speedup vs baseline: 1.4102x; 1.0023x over previous
import functools

import jax
import jax.numpy as jnp
from jax import lax
from jax.experimental import pallas as pl
from jax.experimental.pallas import tpu as pltpu
from jax.experimental.pallas import tpu_sc as plsc

B, L = 4096, 200
DW, DP = 32, 32
DO = DW + DP
N = B * L
NC, NS = 2, 16
NW = NC * NS
BW = B // NW
PER_W = BW * L
NGR = BW
VLANES = 16


def _gather_body(idx_scale, col_off, d,
                 idx_hbm, tbl_hbm, out_hbm, idx_v, rows_v, semg, semo):
    wid = lax.axis_index("s") * NC + lax.axis_index("c")
    base = wid * PER_W
    pltpu.sync_copy(idx_hbm.at[pl.ds(base, PER_W)], idx_v)
    b_base = wid * BW

    if idx_scale != 1:
        def scale(i, carry):
            s = pl.ds(i * VLANES, VLANES)
            idx_v[s] = idx_v[s] * idx_scale
            return carry
        lax.fori_loop(0, PER_W // VLANES, scale, 0)

    def start_gather(s, j):
        pltpu.async_copy(tbl_hbm.at[idx_v.at[pl.ds(j * L, L)]],
                         rows_v.at[s], semg.at[s])

    def wait_gather(s, j):
        pltpu.make_async_copy(tbl_hbm.at[idx_v.at[pl.ds(j * L, L)]],
                              rows_v.at[s], semg.at[s]).wait()

    def start_write(s, j):
        pltpu.async_copy(rows_v.at[s],
                         out_hbm.at[b_base + j, :, pl.ds(col_off, d)],
                         semo.at[s])

    def wait_write(s, j):
        pltpu.make_async_copy(rows_v.at[s],
                              out_hbm.at[b_base + j, :, pl.ds(col_off, d)],
                              semo.at[s]).wait()

    def steps12(s, j):
        wait_gather(s, j)
        start_write(s, j)

    def steps34(s2, j):
        wait_write(s2, j - 2)
        start_gather(s2, j + 2)

    start_gather(0, 0)
    start_gather(1, 1)
    steps12(0, 0)
    start_gather(2, 2)
    steps12(1, 1)
    start_gather(3, 3)
    steps12(2, 2)
    steps34(0, 2)
    steps12(3, 3)
    steps34(1, 3)

    def step(it, carry):
        g = it * 4
        for b in range(4):
            j = g + b
            steps12(b, j)
            steps34((b + 2) % 4, j)
        return carry

    lax.fori_loop(1, NGR // 4 - 1, step, 0)

    for b in range(4):
        j = NGR - 4 + b
        steps12(b, j)
        if j + 2 < NGR:
            steps34((b + 2) % 4, j)
    for b in range(4):
        wait_write(b, NGR - 4 + b)


def _mesh():
    return plsc.VectorSubcoreMesh(
        core_axis_name="c", subcore_axis_name="s",
        num_cores=NC, num_subcores=NS)


def _scratch(d):
    return [
        pltpu.VMEM((PER_W,), jnp.int32),
        pltpu.VMEM((4, L, d), jnp.float32),
        pltpu.SemaphoreType.DMA((4,)),
        pltpu.SemaphoreType.DMA((4,)),
    ]


@jax.jit
def _run(words_f, pos_f, W_words, W_pos):
    kpos = pl.kernel(
        functools.partial(_gather_body, 1, DW, DP),
        out_type=jax.ShapeDtypeStruct((B, L, 128), jnp.float32),
        mesh=_mesh(),
        compiler_params=pltpu.CompilerParams(use_tc_tiling_on_sc=False),
        scratch_types=_scratch(DP),
    )
    out1 = kpos(pos_f, W_pos)
    ref = jax.new_ref(out1)
    W4 = jnp.pad(W_words, ((0, 0), (0, 128 - DW))).reshape(4 * 1000000, DW)
    kwords = pl.kernel(
        functools.partial(_gather_body, 4, 0, DW),
        out_type=(),
        mesh=_mesh(),
        compiler_params=pltpu.CompilerParams(use_tc_tiling_on_sc=False),
        scratch_types=_scratch(DW),
    )
    kwords(words_f, W4, ref)
    return ref[...]


def kernel(words, pos, W_words, W_pos):
    words_f = words.astype(jnp.int32).reshape(N)
    pos_f = pos.astype(jnp.int32).reshape(N)
    out = _run(words_f, pos_f, W_words, W_pos)
    return out[:, :, :DO]
